# initial kernel scaffold (unmeasured)
import jax
import jax.numpy as jnp
from jax import lax
from jax.experimental import pallas as pl
from jax.experimental.pallas import tpu as pltpu

Z = 4


def kernel(x, dy, gamma):
    m, d = x.shape

    def body(x_ref, dy_ref, gamma_ref, out_ref, comm_ref, send_sems, recv_sems):
        my_x = lax.axis_index("x")
        my_y = lax.axis_index("y")
        my_z = lax.axis_index("z")

        xv = x_ref[:, :]
        dyv = dy_ref[:, :]
        mu = jnp.mean(xv, axis=1, keepdims=True)
        xc = xv - mu
        var = jnp.mean(xc * xc, axis=1, keepdims=True)
        rstd = lax.rsqrt(var + 1e-5)
        xhat = xc * rstd
        dgamma = jnp.sum(dyv * xhat, axis=0)[None, :]
        dbeta = jnp.sum(dyv, axis=0)[None, :]
        part = jnp.concatenate([dgamma, dbeta], axis=0)
        comm_ref[pl.ds(my_z, 1)] = part[None]

        barrier_sem = pltpu.get_barrier_semaphore()
        for off in range(1, Z):
            pl.semaphore_signal(
                barrier_sem,
                inc=1,
                device_id=(my_x, my_y, lax.rem(my_z + off, Z)),
                device_id_type=pl.DeviceIdType.MESH,
            )
        pl.semaphore_wait(barrier_sem, Z - 1)

        sends = []
        for off in range(1, Z):
            rdma = pltpu.make_async_remote_copy(
                src_ref=comm_ref.at[my_z],
                dst_ref=comm_ref.at[my_z],
                send_sem=send_sems.at[off - 1],
                recv_sem=recv_sems.at[my_z],
                device_id=(my_x, my_y, lax.rem(my_z + off, Z)),
                device_id_type=pl.DeviceIdType.MESH,
            )
            rdma.start()
            sends.append(rdma)

        for off in range(1, Z):
            src_z = lax.rem(my_z + Z - off, Z)
            recv = pltpu.make_async_remote_copy(
                src_ref=comm_ref.at[src_z],
                dst_ref=comm_ref.at[src_z],
                send_sem=send_sems.at[off - 1],
                recv_sem=recv_sems.at[src_z],
                device_id=(my_x, my_y, my_z),
                device_id_type=pl.DeviceIdType.MESH,
            )
            recv.wait_recv()
        for s in sends:
            s.wait_send()

        out_ref[:, :] = (
            comm_ref[0] + comm_ref[1] + comm_ref[2] + comm_ref[3]
        )

    return pl.pallas_call(
        body,
        out_shape=jax.ShapeDtypeStruct((2, d), jnp.float32),
        in_specs=[
            pl.BlockSpec(memory_space=pltpu.VMEM),
            pl.BlockSpec(memory_space=pltpu.VMEM),
            pl.BlockSpec(memory_space=pltpu.ANY),
        ],
        out_specs=pl.BlockSpec(memory_space=pltpu.VMEM),
        scratch_shapes=[
            pltpu.VMEM((Z, 2, d), jnp.float32),
            pltpu.SemaphoreType.DMA((Z - 1,)),
            pltpu.SemaphoreType.DMA((Z,)),
        ],
        compiler_params=pltpu.CompilerParams(collective_id=0),
    )(x, dy, gamma)


# baseline (device time: 23976 ns/iter reference)
import jax
import jax.numpy as jnp
from jax import lax
from jax.experimental import pallas as pl
from jax.experimental.pallas import tpu as pltpu

Z = 4


def kernel(x, dy, gamma):
    m, d = x.shape

    def body(x_ref, dy_ref, gamma_ref, out_ref, comm_ref, send_sems, recv_sems):
        my_x = lax.axis_index("x")
        my_y = lax.axis_index("y")
        my_z = lax.axis_index("z")

        xv = x_ref[:, :]
        dyv = dy_ref[:, :]
        mu = jnp.mean(xv, axis=1, keepdims=True)
        xc = xv - mu
        var = jnp.mean(xc * xc, axis=1, keepdims=True)
        rstd = lax.rsqrt(var + 1e-5)
        xhat = xc * rstd
        dgamma = jnp.sum(dyv * xhat, axis=0)[None, :]
        dbeta = jnp.sum(dyv, axis=0)[None, :]
        part = jnp.concatenate([dgamma, dbeta], axis=0)
        comm_ref[pl.ds(my_z, 1)] = part[None]

        barrier_sem = pltpu.get_barrier_semaphore()
        for off in range(1, Z):
            pl.semaphore_signal(
                barrier_sem,
                inc=1,
                device_id=(my_x, my_y, lax.rem(my_z + off, Z)),
                device_id_type=pl.DeviceIdType.MESH,
            )
        pl.semaphore_wait(barrier_sem, Z - 1)

        sends = []
        for off in range(1, Z):
            rdma = pltpu.make_async_remote_copy(
                src_ref=comm_ref.at[my_z],
                dst_ref=comm_ref.at[my_z],
                send_sem=send_sems.at[off - 1],
                recv_sem=recv_sems.at[my_z],
                device_id=(my_x, my_y, lax.rem(my_z + off, Z)),
                device_id_type=pl.DeviceIdType.MESH,
            )
            rdma.start()
            sends.append(rdma)

        for off in range(1, Z):
            src_z = lax.rem(my_z + Z - off, Z)
            recv = pltpu.make_async_remote_copy(
                src_ref=comm_ref.at[src_z],
                dst_ref=comm_ref.at[src_z],
                send_sem=send_sems.at[off - 1],
                recv_sem=recv_sems.at[src_z],
                device_id=(my_x, my_y, my_z),
                device_id_type=pl.DeviceIdType.MESH,
            )
            recv.wait_recv()
        for s in sends:
            s.wait_send()

        out_ref[:, :] = (
            comm_ref[0] + comm_ref[1] + comm_ref[2] + comm_ref[3]
        )

    return pl.pallas_call(
        body,
        out_shape=jax.ShapeDtypeStruct((2, d), jnp.float32),
        in_specs=[
            pl.BlockSpec(memory_space=pltpu.VMEM),
            pl.BlockSpec(memory_space=pltpu.VMEM),
            pl.BlockSpec(memory_space=pl.ANY),
        ],
        out_specs=pl.BlockSpec(memory_space=pltpu.VMEM),
        scratch_shapes=[
            pltpu.VMEM((Z, 2, d), jnp.float32),
            pltpu.SemaphoreType.DMA((Z - 1,)),
            pltpu.SemaphoreType.DMA((Z,)),
        ],
        compiler_params=pltpu.CompilerParams(collective_id=0),
    )(x, dy, gamma)


# device time: 23587 ns/iter; 1.0165x vs baseline; 1.0165x over previous
import jax
import jax.numpy as jnp
from jax import lax
from jax.experimental import pallas as pl
from jax.experimental.pallas import tpu as pltpu

Z = 4
BLK = 256


def kernel(x, dy, gamma):
    m, d = x.shape
    grid = m // BLK

    def body(x_ref, dy_ref, gamma_ref, out_ref, acc_ref, comm_ref,
             send_sems, recv_sems):
        step = pl.program_id(0)

        xv = x_ref[:, :]
        dyv = dy_ref[:, :]
        mu = jnp.mean(xv, axis=1, keepdims=True)
        xc = xv - mu
        var = jnp.mean(xc * xc, axis=1, keepdims=True)
        rstd = lax.rsqrt(var + 1e-5)
        xhat = xc * rstd
        dgamma = jnp.sum(dyv * xhat, axis=0)[None, :]
        dbeta = jnp.sum(dyv, axis=0)[None, :]
        part = jnp.concatenate([dgamma, dbeta], axis=0)

        @pl.when(step == 0)
        def _():
            acc_ref[:, :] = part

        @pl.when(step != 0)
        def _():
            acc_ref[:, :] = acc_ref[:, :] + part

        @pl.when(step == grid - 1)
        def _():
            my_x = lax.axis_index("x")
            my_y = lax.axis_index("y")
            my_z = lax.axis_index("z")

            comm_ref[pl.ds(my_z, 1)] = acc_ref[:, :][None]

            barrier_sem = pltpu.get_barrier_semaphore()
            for off in range(1, Z):
                pl.semaphore_signal(
                    barrier_sem,
                    inc=1,
                    device_id=(my_x, my_y, lax.rem(my_z + off, Z)),
                    device_id_type=pl.DeviceIdType.MESH,
                )
            pl.semaphore_wait(barrier_sem, Z - 1)

            sends = []
            for off in range(1, Z):
                rdma = pltpu.make_async_remote_copy(
                    src_ref=comm_ref.at[my_z],
                    dst_ref=comm_ref.at[my_z],
                    send_sem=send_sems.at[off - 1],
                    recv_sem=recv_sems.at[my_z],
                    device_id=(my_x, my_y, lax.rem(my_z + off, Z)),
                    device_id_type=pl.DeviceIdType.MESH,
                )
                rdma.start()
                sends.append(rdma)

            for off in range(1, Z):
                src_z = lax.rem(my_z + Z - off, Z)
                recv = pltpu.make_async_remote_copy(
                    src_ref=comm_ref.at[src_z],
                    dst_ref=comm_ref.at[src_z],
                    send_sem=send_sems.at[off - 1],
                    recv_sem=recv_sems.at[src_z],
                    device_id=(my_x, my_y, my_z),
                    device_id_type=pl.DeviceIdType.MESH,
                )
                recv.wait_recv()
            for s in sends:
                s.wait_send()

            out_ref[:, :] = (
                comm_ref[0] + comm_ref[1] + comm_ref[2] + comm_ref[3]
            )

    return pl.pallas_call(
        body,
        grid=(grid,),
        out_shape=jax.ShapeDtypeStruct((2, d), jnp.float32),
        in_specs=[
            pl.BlockSpec((BLK, d), lambda i: (i, 0)),
            pl.BlockSpec((BLK, d), lambda i: (i, 0)),
            pl.BlockSpec(memory_space=pl.ANY),
        ],
        out_specs=pl.BlockSpec((2, d), lambda i: (0, 0)),
        scratch_shapes=[
            pltpu.VMEM((2, d), jnp.float32),
            pltpu.VMEM((Z, 2, d), jnp.float32),
            pltpu.SemaphoreType.DMA((Z - 1,)),
            pltpu.SemaphoreType.DMA((Z,)),
        ],
        compiler_params=pltpu.CompilerParams(collective_id=0),
    )(x, dy, gamma)
